# baseline (speedup 1.0000x reference)
import functools

import jax
import jax.numpy as jnp
import numpy as np
from jax import lax
from jax.experimental import pallas as pl
from jax.experimental.pallas import tpu as pltpu
from jax.experimental.pallas import tpu_sc as plsc

_I32_MIN = np.int32(-2147483648)
_I32_LOW = np.int32(2147483647)
_NBLK = 128


def _sc_select_body(R, L, x_hbm, theta_hbm, out_hbm, rows_v, hist_v,
                    sums_v, theta_v, thr_v, sem0, sem1):
    nsub = 16
    wid = lax.axis_index("s") * 2 + lax.axis_index("c")
    r0 = wid * 2

    cp0 = pltpu.async_copy(x_hbm.at[r0], rows_v.at[0], sem0)
    cp1 = pltpu.async_copy(x_hbm.at[r0 + 1], rows_v.at[1], sem1)

    pltpu.sync_copy(theta_hbm, theta_v)
    th = theta_v[...]
    act = 1.0 / (1.0 + jnp.exp(-th))
    kf = L * (1.0 - act)
    k = jnp.clip(kf.astype(jnp.int32), 1, L)

    ones = jnp.full((nsub,), 1, jnp.int32)
    last = jnp.full((nsub,), nsub - 1, jnp.int32)
    lane0 = lax.iota(jnp.int32, nsub) == 0
    zero_v = jnp.zeros((nsub,), jnp.int32)

    def splat_last(v):
        return v.at[last].get(mode="promise_in_bounds")

    def blk(i, raw_order):
        if not raw_order:
            return i * nsub, None
        base = jnp.where(i < 64, 2032 - i * nsub, i * nsub - 1024)
        condv = jnp.full((nsub,), i, jnp.int32) < 64
        return base, condv

    def load_u(i, raw_order):
        base, condv = blk(i, raw_order)
        h = hist_v[pl.ds(base, nsub)]
        if raw_order:
            h = jnp.where(condv, lax.rev(h, (0,)), h)
        return h, base

    def scan(rbud, raw_order):
        @plsc.parallel_loop(0, _NBLK, unroll=2)
        def _(i):
            h, _ = load_u(i, raw_order)
            cs = plsc.cumsum(h)
            plsc.store_scatter(sums_v, [jnp.full((nsub,), i, jnp.int32)],
                               splat_last(cs), mask=lane0)

        c = zero_v
        for sb in range(_NBLK // nsub):
            s = sums_v[pl.ds(sb * nsub, nsub)]
            ps = plsc.cumsum(s) + c
            sums_v[pl.ds(sb * nsub, nsub)] = ps - s
            c = splat_last(ps)

        @plsc.parallel_loop(0, _NBLK, unroll=2, carry=(zero_v, zero_v))
        def scarry(i, carry):
            bacc, mvec = carry
            h, base = load_u(i, raw_order)
            cs = plsc.cumsum(h) + plsc.load_gather(
                sums_v, [jnp.full((nsub,), i, jnp.int32)])
            m = cs <= rbud
            bacc = bacc + plsc.all_reduce_population_count(m)
            mvec = jnp.maximum(mvec, jnp.where(m, cs, 0))
            hist_v[pl.ds(base, nsub)] = zero_v
            return bacc, mvec

        bacc, mvec = scarry
        return bacc, splat_last(plsc.cummax(mvec))

    @plsc.parallel_loop(0, _NBLK, unroll=4)
    def _(i):
        hist_v[pl.ds(i * nsub, nsub)] = zero_v

    waits = [cp0.wait, cp1.wait]
    row_thr = []

    for j in range(2):
        waits[j]()

        @plsc.parallel_loop(0, L // nsub, unroll=8)
        def _(i):
            v = rows_v[j, pl.ds(i * nsub, nsub)]
            bits = lax.bitcast_convert_type(v, jnp.int32)
            plsc.addupdate_scatter(
                hist_v, [lax.shift_right_logical(bits, 21)], ones)

        rbud1 = jnp.int32(L) - k
        b1, m1 = scan(rbud1, raw_order=True)

        neg = b1 < 1024
        sgnv = jnp.where(neg, jnp.full((nsub,), -1, jnp.int32),
                         jnp.full((nsub,), _I32_MIN, jnp.int32))
        basev = b1 << 11
        rbud2 = rbud1 - m1

        @plsc.parallel_loop(0, L // nsub, unroll=8)
        def _(i):
            v = rows_v[j, pl.ds(i * nsub, nsub)]
            bits = lax.bitcast_convert_type(v, jnp.int32)
            t = lax.shift_right_logical(bits ^ sgnv, 10)
            d = t - basev
            m = plsc.bitcast(d, jnp.uint32) < jnp.uint32(2048)
            plsc.addupdate_scatter(hist_v, [d], ones, mask=m)

        b2, _ = scan(rbud2, raw_order=False)

        qv = ((b1 << 11) | b2) << 10
        q_s = qv ^ _I32_MIN
        fbits = jnp.where(q_s < 0, q_s ^ _I32_LOW, q_s)
        row_thr.append(lax.bitcast_convert_type(fbits, jnp.float32))

    thr_v[...] = jnp.minimum(row_thr[0], row_thr[1])
    pltpu.sync_copy(thr_v, out_hbm.at[wid])


def _sc_select(inputs, theta):
    R, L = inputs.shape
    mesh = plsc.VectorSubcoreMesh(core_axis_name="c", subcore_axis_name="s")
    kfn = functools.partial(
        pl.kernel,
        mesh=mesh,
        compiler_params=pltpu.CompilerParams(
            needs_layout_passes=False,
            disable_bounds_checks=True,
        ),
        out_type=jax.ShapeDtypeStruct((32, 16), jnp.float32),
        scratch_types=[
            pltpu.VMEM((2, L), jnp.float32),
            pltpu.VMEM((2048,), jnp.int32),
            pltpu.VMEM((128,), jnp.int32),
            pltpu.VMEM((16,), jnp.float32),
            pltpu.VMEM((16,), jnp.float32),
            pltpu.SemaphoreType.DMA,
            pltpu.SemaphoreType.DMA,
        ],
    )(functools.partial(_sc_select_body, R, L))
    return kfn(inputs, jnp.broadcast_to(theta, (16,)))


def _tc_mask_body(thr_ref, x_ref, o_ref):
    mid = jnp.min(thr_ref[...])
    z = x_ref[...] - mid
    ez = jnp.exp(-jnp.abs(z))
    t = 1.0 / (1.0 + ez)
    o_ref[...] = jnp.where(z >= 0, t, 1.0 - t)


def _tc_mask(inputs, thr):
    R, L = inputs.shape
    blk = 4096
    return pl.pallas_call(
        _tc_mask_body,
        out_shape=jax.ShapeDtypeStruct((R, L), jnp.float32),
        grid=(L // blk,),
        in_specs=[
            pl.BlockSpec((32, 16), lambda i: (0, 0)),
            pl.BlockSpec((R, blk), lambda i: (0, i)),
        ],
        out_specs=pl.BlockSpec((R, blk), lambda i: (0, i)),
    )(thr, inputs)


def kernel(inputs, theta):
    thr = _sc_select(inputs, theta)
    return _tc_mask(inputs, thr)


# speedup vs baseline: 1.0092x; 1.0092x over previous
import functools

import jax
import jax.numpy as jnp
import numpy as np
from jax import lax
from jax.experimental import pallas as pl
from jax.experimental.pallas import tpu as pltpu
from jax.experimental.pallas import tpu_sc as plsc

_I32_MIN = np.int32(-2147483648)
_I32_LOW = np.int32(2147483647)
_NBLK = 128


def _sc_select_body(R, L, x_hbm, theta_hbm, out_hbm, rows_v, hist_v,
                    sums_v, theta_v, thr_v, sem0, sem1):
    nsub = 16
    wid = lax.axis_index("s") * 2 + lax.axis_index("c")
    r0 = wid * 2

    cp0 = pltpu.async_copy(x_hbm.at[r0], rows_v.at[0], sem0)
    cp1 = pltpu.async_copy(x_hbm.at[r0 + 1], rows_v.at[1], sem1)

    pltpu.sync_copy(theta_hbm, theta_v)
    th = theta_v[...]
    act = 1.0 / (1.0 + jnp.exp(-th))
    kf = L * (1.0 - act)
    k = jnp.clip(kf.astype(jnp.int32), 1, L)

    ones = jnp.full((nsub,), 1, jnp.int32)
    last = jnp.full((nsub,), nsub - 1, jnp.int32)
    lane0 = lax.iota(jnp.int32, nsub) == 0
    zero_v = jnp.zeros((nsub,), jnp.int32)

    def splat_last(v):
        return v.at[last].get(mode="promise_in_bounds")

    def blk(i, raw_order):
        if not raw_order:
            return i * nsub, None
        base = jnp.where(i < 64, 2032 - i * nsub, i * nsub - 1024)
        condv = jnp.full((nsub,), i, jnp.int32) < 64
        return base, condv

    def load_u(i, raw_order):
        base, condv = blk(i, raw_order)
        h = hist_v[pl.ds(base, nsub)]
        if raw_order:
            h = jnp.where(condv, lax.rev(h, (0,)), h)
        return h, base

    def scan(rbud, raw_order):
        @plsc.parallel_loop(0, _NBLK, carry=(zero_v, zero_v, zero_v))
        def scarry(i, carry):
            c, bacc, mvec = carry
            h, base = load_u(i, raw_order)
            cs = plsc.cumsum(h) + c
            m = cs <= rbud
            bacc = bacc + plsc.all_reduce_population_count(m)
            mvec = jnp.maximum(mvec, jnp.where(m, cs, 0))
            hist_v[pl.ds(base, nsub)] = zero_v
            return splat_last(cs), bacc, mvec

        _, bacc, mvec = scarry
        return bacc, splat_last(plsc.cummax(mvec))

    @plsc.parallel_loop(0, _NBLK, unroll=4)
    def _(i):
        hist_v[pl.ds(i * nsub, nsub)] = zero_v

    waits = [cp0.wait, cp1.wait]
    row_thr = []

    for j in range(2):
        waits[j]()

        @plsc.parallel_loop(0, L // nsub, unroll=4)
        def _(i):
            v = rows_v[j, pl.ds(i * nsub, nsub)]
            bits = lax.bitcast_convert_type(v, jnp.int32)
            plsc.addupdate_scatter(
                hist_v, [lax.shift_right_logical(bits, 21)], ones)

        rbud1 = jnp.int32(L) - k
        b1, m1 = scan(rbud1, raw_order=True)

        neg = b1 < 1024
        sgnv = jnp.where(neg, jnp.full((nsub,), -1, jnp.int32),
                         jnp.full((nsub,), _I32_MIN, jnp.int32))
        basev = b1 << 11
        rbud2 = rbud1 - m1

        @plsc.parallel_loop(0, L // nsub, unroll=4)
        def _(i):
            v = rows_v[j, pl.ds(i * nsub, nsub)]
            bits = lax.bitcast_convert_type(v, jnp.int32)
            t = lax.shift_right_logical(bits ^ sgnv, 10)
            d = t - basev
            m = plsc.bitcast(d, jnp.uint32) < jnp.uint32(2048)
            plsc.addupdate_scatter(hist_v, [d], ones, mask=m)

        b2, _ = scan(rbud2, raw_order=False)

        qv = ((b1 << 11) | b2) << 10
        q_s = qv ^ _I32_MIN
        fbits = jnp.where(q_s < 0, q_s ^ _I32_LOW, q_s)
        row_thr.append(lax.bitcast_convert_type(fbits, jnp.float32))

    thr_v[...] = jnp.minimum(row_thr[0], row_thr[1])
    pltpu.sync_copy(thr_v, out_hbm.at[wid])


def _sc_select(inputs, theta):
    R, L = inputs.shape
    mesh = plsc.VectorSubcoreMesh(core_axis_name="c", subcore_axis_name="s")
    kfn = functools.partial(
        pl.kernel,
        mesh=mesh,
        compiler_params=pltpu.CompilerParams(
            needs_layout_passes=False,
            disable_bounds_checks=True,
        ),
        out_type=jax.ShapeDtypeStruct((32, 16), jnp.float32),
        scratch_types=[
            pltpu.VMEM((2, L), jnp.float32),
            pltpu.VMEM((2048,), jnp.int32),
            pltpu.VMEM((128,), jnp.int32),
            pltpu.VMEM((16,), jnp.float32),
            pltpu.VMEM((16,), jnp.float32),
            pltpu.SemaphoreType.DMA,
            pltpu.SemaphoreType.DMA,
        ],
    )(functools.partial(_sc_select_body, R, L))
    return kfn(inputs, jnp.broadcast_to(theta, (16,)))


def _tc_mask_body(thr_ref, x_ref, o_ref):
    mid = jnp.min(thr_ref[...])
    z = x_ref[...] - mid
    ez = jnp.exp(-jnp.abs(z))
    t = 1.0 / (1.0 + ez)
    o_ref[...] = jnp.where(z >= 0, t, 1.0 - t)


def _tc_mask(inputs, thr):
    R, L = inputs.shape
    blk = 4096
    return pl.pallas_call(
        _tc_mask_body,
        out_shape=jax.ShapeDtypeStruct((R, L), jnp.float32),
        grid=(L // blk,),
        in_specs=[
            pl.BlockSpec((32, 16), lambda i: (0, 0)),
            pl.BlockSpec((R, blk), lambda i: (0, i)),
        ],
        out_specs=pl.BlockSpec((R, blk), lambda i: (0, i)),
    )(thr, inputs)


def kernel(inputs, theta):
    thr = _sc_select(inputs, theta)
    return _tc_mask(inputs, thr)
